# baseline (speedup 1.0000x reference)
import functools

import jax
import jax.numpy as jnp
from jax import lax
from jax.experimental import pallas as pl
from jax.experimental.pallas import tpu as pltpu
from jax.experimental.pallas import tpu_sc as plsc

_CHUNK = 320
_NBUF = 4


@functools.lru_cache(maxsize=None)
def _make_gather(B, V, D):
    info = plsc.get_sparse_core_info()
    nc, ns = info.num_cores, info.num_subcores
    nw = nc * ns
    assert B % (nw * _CHUNK) == 0
    b_per_w = B // nw
    n_chunks = b_per_w // _CHUNK
    assert n_chunks >= _NBUF
    mesh = plsc.VectorSubcoreMesh(core_axis_name="c", subcore_axis_name="s")

    @functools.partial(
        pl.kernel,
        out_type=jax.ShapeDtypeStruct((B, D), jnp.float32),
        mesh=mesh,
        scratch_types=[
            pltpu.VMEM((b_per_w,), jnp.int32),
        ]
        + [pltpu.VMEM((_CHUNK, D), jnp.float32) for _ in range(_NBUF)]
        + [
            pltpu.SemaphoreType.DMA,
            pltpu.SemaphoreType.DMA,
        ],
        compiler_params=pltpu.CompilerParams(use_tc_tiling_on_sc=False),
    )
    def gather_kernel(table_hbm, idx_hbm, out_hbm, idx_v, *rest):
        bufs = rest[:_NBUF]
        gsem, osem = rest[_NBUF], rest[_NBUF + 1]
        wid = lax.axis_index("s") * nc + lax.axis_index("c")
        base = wid * b_per_w
        pltpu.sync_copy(idx_hbm.at[pl.ds(base, b_per_w)], idx_v)

        def gather_start(g, rows):
            pltpu.async_copy(
                table_hbm.at[idx_v.at[pl.ds(g * _CHUNK, _CHUNK)]], rows, gsem
            )

        def write_start(g, rows):
            pltpu.async_copy(rows, out_hbm.at[pl.ds(base + g * _CHUNK, _CHUNK)], osem)

        def drain_one(sem):
            pltpu.make_async_copy(table_hbm.at[pl.ds(0, _CHUNK)], bufs[0], sem).wait()

        def on_buf(m, fn):
            for i in range(_NBUF):
                @pl.when(m == i)
                def _(r=bufs[i]):
                    fn(r)

        for g in range(_NBUF - 1):
            gather_start(g, bufs[g])

        def step(g, carry):
            @pl.when(g + _NBUF - 1 < n_chunks)
            def _():
                @pl.when(g >= 1)
                def _():
                    drain_one(osem)

                on_buf((g + _NBUF - 1) % _NBUF,
                       lambda r: gather_start(g + _NBUF - 1, r))

            drain_one(gsem)
            on_buf(g % _NBUF, lambda r: write_start(g, r))
            return carry

        lax.fori_loop(0, n_chunks, step, 0)
        for _ in range(_NBUF):
            drain_one(osem)

    return gather_kernel


def kernel(token_ids, embedding):
    V, D = embedding.shape
    B = token_ids.shape[0] * token_ids.shape[1]
    idx = token_ids.reshape(-1).astype(jnp.int32)
    out = _make_gather(B, V, D)(embedding, idx)
    return out.reshape(token_ids.shape + (D,))
